# Initial kernel scaffold; baseline (speedup 1.0000x reference)
import jax
import jax.numpy as jnp
from jax.experimental import pallas as pl


def kernel(x, edge_index, W0, b0, Wl1, bl1, Wr1, Wl2, bl2, Wr2, W1, b1):
    raise NotImplementedError("write your pallas kernel here")


# speedup vs baseline: 6.9095x; 6.9095x over previous
import functools

import jax
import jax.numpy as jnp
from jax import lax
from jax.experimental import pallas as pl
from jax.experimental.pallas import tpu as pltpu
import jax.experimental.pallas.tpu_sc as plsc

N = 10000
D = 128
E = 320000

NC = 2
NS = 16
NW = NC * NS
EPW = E // NW
CHUNK = 80
NCHUNK = EPW // CHUNK
NPAD = 10240
RPT = NPAD // NS
ZR = 128
CPT = NPAD // NS


def _seg_body(with_counts, h_hbm, src_hbm, dst_hbm, part_hbm, *rest):
    if with_counts:
        (cnt_hbm, src_v, dst_v, dstc_v, rows_v, zbuf, acc_s, cnt_s, ones_v,
         cbuf, gsem) = rest
    else:
        src_v, dst_v, dstc_v, rows_v, zbuf, acc_s, gsem = rest
    c = lax.axis_index("c")
    s = lax.axis_index("s")
    w = c * NS + s

    pltpu.sync_copy(src_hbm.at[pl.ds(w * EPW, EPW)], src_v)
    pltpu.sync_copy(dst_hbm.at[pl.ds(w * EPW, EPW)], dst_v)

    def _zrow(i, carry):
        def _zlane(j, carry2):
            zbuf[i, pl.ds(j * 16, 16)] = jnp.zeros((16,), jnp.float32)
            return carry2
        return lax.fori_loop(0, D // 16, _zlane, carry)
    lax.fori_loop(0, ZR, _zrow, 0)
    for t in range(RPT // ZR):
        pltpu.sync_copy(zbuf, acc_s.at[pl.ds(s * RPT + t * ZR, ZR), :])

    if with_counts:
        def _zc(i, carry):
            cbuf[pl.ds(i * 16, 16)] = jnp.zeros((16,), jnp.float32)
            return carry
        lax.fori_loop(0, CPT // 16, _zc, 0)
        pltpu.sync_copy(cbuf, cnt_s.at[pl.ds(s * CPT, CPT)])
        for j in range(CHUNK // 16):
            ones_v[pl.ds(j * 16, 16)] = jnp.ones((16,), jnp.float32)

    plsc.subcore_barrier()

    def _edge(j, carry):
        gcp = pltpu.async_copy(h_hbm.at[src_v.at[pl.ds(j * CHUNK, CHUNK)]],
                               rows_v, gsem)
        for k in range(CHUNK // 16):
            dstc_v[pl.ds(k * 16, 16)] = dst_v[pl.ds(j * CHUNK + k * 16, 16)]
        gcp.wait()
        pltpu.sync_copy(rows_v, acc_s.at[dstc_v], add=True)
        if with_counts:
            pltpu.sync_copy(ones_v, cnt_s.at[dstc_v], add=True)
        return carry
    lax.fori_loop(0, NCHUNK, _edge, 0)

    plsc.subcore_barrier()

    for t in range(RPT // ZR):
        pltpu.sync_copy(acc_s.at[pl.ds(s * RPT + t * ZR, ZR), :], zbuf)
        pltpu.sync_copy(zbuf,
                        part_hbm.at[c, pl.ds(s * RPT + t * ZR, ZR), :])
    if with_counts:
        pltpu.sync_copy(cnt_s.at[pl.ds(s * CPT, CPT)], cbuf)
        pltpu.sync_copy(cbuf, cnt_hbm.at[pl.ds(c * NPAD + s * CPT, CPT)])


def _make_seg(with_counts):
    out_type = [jax.ShapeDtypeStruct((NC, NPAD, D), jnp.float32)]
    scratch = [
        pltpu.VMEM((EPW,), jnp.int32),
        pltpu.VMEM((EPW,), jnp.int32),
        pltpu.VMEM((CHUNK,), jnp.int32),
        pltpu.VMEM((CHUNK, D), jnp.float32),
        pltpu.VMEM((ZR, D), jnp.float32),
        pltpu.VMEM_SHARED((NPAD, D), jnp.float32),
    ]
    if with_counts:
        out_type.append(jax.ShapeDtypeStruct((NC * NPAD,), jnp.float32))
        scratch += [
            pltpu.VMEM_SHARED((NPAD,), jnp.float32),
            pltpu.VMEM((CHUNK,), jnp.float32),
            pltpu.VMEM((CPT,), jnp.float32),
        ]
    scratch.append(pltpu.SemaphoreType.DMA)
    return pl.kernel(
        functools.partial(_seg_body, with_counts),
        out_type=out_type,
        mesh=plsc.VectorSubcoreMesh(core_axis_name="c", subcore_axis_name="s"),
        scratch_types=scratch,
    )


_seg_with_counts = _make_seg(True)
_seg_no_counts = _make_seg(False)


TBLK = 1000


def _t0_body(x_ref, w_ref, b_ref, o_ref):
    o_ref[...] = jnp.maximum(
        jnp.dot(x_ref[...], w_ref[...], preferred_element_type=jnp.float32)
        + b_ref[...], 0.0)


def _layer_body(p0_ref, p1_ref, c0_ref, c1_ref, h_ref, wl_ref, bl_ref,
                wr_ref, o_ref):
    cnt = jnp.maximum(c0_ref[...] + c1_ref[...], 1.0)
    agg = (p0_ref[...] + p1_ref[...]) / cnt
    o_ref[...] = jnp.maximum(
        jnp.dot(agg, wl_ref[...], preferred_element_type=jnp.float32)
        + bl_ref[...]
        + jnp.dot(h_ref[...], wr_ref[...], preferred_element_type=jnp.float32),
        0.0)


def _layer_final_body(p0_ref, p1_ref, c0_ref, c1_ref, h_ref, wl_ref, bl_ref,
                      wr_ref, w1_ref, b1_ref, o_ref):
    cnt = jnp.maximum(c0_ref[...] + c1_ref[...], 1.0)
    agg = (p0_ref[...] + p1_ref[...]) / cnt
    h2 = jnp.maximum(
        jnp.dot(agg, wl_ref[...], preferred_element_type=jnp.float32)
        + bl_ref[...]
        + jnp.dot(h_ref[...], wr_ref[...], preferred_element_type=jnp.float32),
        0.0)
    o_ref[...] = (jnp.dot(h2, w1_ref[...], preferred_element_type=jnp.float32)
                  + b1_ref[...])


_row_spec = pl.BlockSpec((TBLK, D), lambda i: (i, 0))
_w_spec = pl.BlockSpec((D, D), lambda i: (0, 0))
_b_spec = pl.BlockSpec((1, D), lambda i: (0, 0))
_c_spec = pl.BlockSpec((TBLK, 1), lambda i: (i, 0))
_out_row = jax.ShapeDtypeStruct((N, D), jnp.float32)
_grid = (N // TBLK,)

_t0 = pl.pallas_call(
    _t0_body, grid=_grid,
    in_specs=[_row_spec, _w_spec, _b_spec],
    out_specs=_row_spec, out_shape=_out_row)

_layer = pl.pallas_call(
    _layer_body, grid=_grid,
    in_specs=[_row_spec, _row_spec, _c_spec, _c_spec, _row_spec,
              _w_spec, _b_spec, _w_spec],
    out_specs=_row_spec, out_shape=_out_row)

_layer_final = pl.pallas_call(
    _layer_final_body, grid=_grid,
    in_specs=[_row_spec, _row_spec, _c_spec, _c_spec, _row_spec,
              _w_spec, _b_spec, _w_spec, _w_spec, _b_spec],
    out_specs=_row_spec, out_shape=_out_row)


def kernel(x, edge_index, W0, b0, Wl1, bl1, Wr1, Wl2, bl2, Wr2, W1, b1):
    src = edge_index[0]
    dst = edge_index[1]
    b0r = b0.reshape(1, D)
    bl1r = bl1.reshape(1, D)
    bl2r = bl2.reshape(1, D)
    b1r = b1.reshape(1, D)

    h0 = _t0(x, W0, b0r)
    part1, cnt = _seg_with_counts(h0, src, dst)
    cnt2 = cnt.reshape(NC, NPAD)
    c0 = cnt2[0, :N].reshape(N, 1)
    c1 = cnt2[1, :N].reshape(N, 1)
    h1 = _layer(part1[0, :N], part1[1, :N], c0, c1, h0, Wl1, bl1r, Wr1)
    (part2,) = _seg_no_counts(h1, src, dst)
    out = _layer_final(part2[0, :N], part2[1, :N], c0, c1, h1, Wl2, bl2r, Wr2,
                       W1, b1r)
    return out


# speedup vs baseline: 10.6715x; 1.5445x over previous
import functools

import jax
import jax.numpy as jnp
from jax import lax
from jax.experimental import pallas as pl
from jax.experimental.pallas import tpu as pltpu
import jax.experimental.pallas.tpu_sc as plsc

N = 10000
D = 128
E = 320000

NC = 2
NS = 16
NW = NC * NS
EPW = E // NW
CHUNK = 80
NCHUNK = EPW // CHUNK
NPAD = 10240
RPT = NPAD // NS
ZR = 128
CPT = NPAD // NS


def _seg_body(with_counts, h_hbm, src_hbm, dst_hbm, part_hbm, *rest):
    if with_counts:
        (cnt_hbm, src_v, dst_v, dstc_v, rbuf0, rbuf1, acc_s, cnt_s,
         ones_v, cbuf, gsem0, gsem1) = rest
    else:
        src_v, dst_v, dstc_v, rbuf0, rbuf1, acc_s, gsem0, gsem1 = rest
    c = lax.axis_index("c")
    s = lax.axis_index("s")
    w = c * NS + s

    pltpu.sync_copy(src_hbm.at[pl.ds(w * EPW, EPW)], src_v)
    pltpu.sync_copy(dst_hbm.at[pl.ds(w * EPW, EPW)], dst_v)

    def _zrow(i, carry):
        def _zlane(j, carry2):
            rbuf0[i, pl.ds(j * 16, 16)] = jnp.zeros((16,), jnp.float32)
            return carry2
        return lax.fori_loop(0, D // 16, _zlane, carry)
    lax.fori_loop(0, CHUNK, _zrow, 0)
    for t in range(RPT // CHUNK):
        pltpu.sync_copy(rbuf0, acc_s.at[pl.ds(s * RPT + t * CHUNK, CHUNK), :])

    if with_counts:
        def _zc(i, carry):
            cbuf[pl.ds(i * 16, 16)] = jnp.zeros((16,), jnp.float32)
            return carry
        lax.fori_loop(0, CPT // 16, _zc, 0)
        pltpu.sync_copy(cbuf, cnt_s.at[pl.ds(s * CPT, CPT)])
        for j in range(CHUNK // 16):
            ones_v[pl.ds(j * 16, 16)] = jnp.ones((16,), jnp.float32)

    plsc.subcore_barrier()

    def _start(i, rb, sem):
        pltpu.async_copy(h_hbm.at[src_v.at[pl.ds(i * CHUNK, CHUNK)]], rb, sem)

    def _consume(i, rb, sem, prefetch_i):
        pltpu.make_async_copy(h_hbm.at[src_v.at[pl.ds(0, CHUNK)]], rb,
                              sem).wait()
        for k in range(CHUNK // 16):
            dstc_v[pl.ds(k * 16, 16)] = dst_v[pl.ds(i * CHUNK + k * 16, 16)]
        pltpu.sync_copy(rb, acc_s.at[dstc_v], add=True)
        if with_counts:
            pltpu.sync_copy(ones_v, cnt_s.at[dstc_v], add=True)

        @pl.when(prefetch_i < NCHUNK)
        def _():
            _start(prefetch_i, rb, sem)

    _start(0, rbuf0, gsem0)
    _start(1, rbuf1, gsem1)

    def _edge2(k, carry):
        i = 2 * k
        _consume(i, rbuf0, gsem0, i + 2)
        _consume(i + 1, rbuf1, gsem1, i + 3)
        return carry
    lax.fori_loop(0, NCHUNK // 2, _edge2, 0)
    _consume(NCHUNK - 1, rbuf0, gsem0, NCHUNK)

    plsc.subcore_barrier()

    for t in range(RPT // CHUNK):
        pltpu.sync_copy(acc_s.at[pl.ds(s * RPT + t * CHUNK, CHUNK), :], rbuf0)
        pltpu.sync_copy(rbuf0,
                        part_hbm.at[c, pl.ds(s * RPT + t * CHUNK, CHUNK), :])
    if with_counts:
        pltpu.sync_copy(cnt_s.at[pl.ds(s * CPT, CPT)], cbuf)
        pltpu.sync_copy(cbuf, cnt_hbm.at[pl.ds(c * NPAD + s * CPT, CPT)])


def _make_seg(with_counts):
    out_type = [jax.ShapeDtypeStruct((NC, NPAD, D), jnp.float32)]
    scratch = [
        pltpu.VMEM((EPW,), jnp.int32),
        pltpu.VMEM((EPW,), jnp.int32),
        pltpu.VMEM((CHUNK,), jnp.int32),
        pltpu.VMEM((CHUNK, D), jnp.float32),
        pltpu.VMEM((CHUNK, D), jnp.float32),
        pltpu.VMEM_SHARED((NPAD, D), jnp.float32),
    ]
    if with_counts:
        out_type.append(jax.ShapeDtypeStruct((NC * NPAD,), jnp.float32))
        scratch += [
            pltpu.VMEM_SHARED((NPAD,), jnp.float32),
            pltpu.VMEM((CHUNK,), jnp.float32),
            pltpu.VMEM((CPT,), jnp.float32),
        ]
    scratch.append(pltpu.SemaphoreType.DMA)
    scratch.append(pltpu.SemaphoreType.DMA)
    return pl.kernel(
        functools.partial(_seg_body, with_counts),
        out_type=out_type,
        mesh=plsc.VectorSubcoreMesh(core_axis_name="c", subcore_axis_name="s"),
        scratch_types=scratch,
    )


_seg_with_counts = _make_seg(True)
_seg_no_counts = _make_seg(False)


TBLK = 1000


def _t0_body(x_ref, w_ref, b_ref, o_ref):
    o_ref[...] = jnp.maximum(
        jnp.dot(x_ref[...], w_ref[...], preferred_element_type=jnp.float32)
        + b_ref[...], 0.0)


def _layer_body(p0_ref, p1_ref, c0_ref, c1_ref, h_ref, wl_ref, bl_ref,
                wr_ref, o_ref):
    cnt = jnp.maximum(c0_ref[...] + c1_ref[...], 1.0)
    agg = (p0_ref[...] + p1_ref[...]) / cnt
    o_ref[...] = jnp.maximum(
        jnp.dot(agg, wl_ref[...], preferred_element_type=jnp.float32)
        + bl_ref[...]
        + jnp.dot(h_ref[...], wr_ref[...], preferred_element_type=jnp.float32),
        0.0)


def _layer_final_body(p0_ref, p1_ref, c0_ref, c1_ref, h_ref, wl_ref, bl_ref,
                      wr_ref, w1_ref, b1_ref, o_ref):
    cnt = jnp.maximum(c0_ref[...] + c1_ref[...], 1.0)
    agg = (p0_ref[...] + p1_ref[...]) / cnt
    h2 = jnp.maximum(
        jnp.dot(agg, wl_ref[...], preferred_element_type=jnp.float32)
        + bl_ref[...]
        + jnp.dot(h_ref[...], wr_ref[...], preferred_element_type=jnp.float32),
        0.0)
    o_ref[...] = (jnp.dot(h2, w1_ref[...], preferred_element_type=jnp.float32)
                  + b1_ref[...])


_row_spec = pl.BlockSpec((TBLK, D), lambda i: (i, 0))
_w_spec = pl.BlockSpec((D, D), lambda i: (0, 0))
_b_spec = pl.BlockSpec((1, D), lambda i: (0, 0))
_c_spec = pl.BlockSpec((TBLK, 1), lambda i: (i, 0))
_out_row = jax.ShapeDtypeStruct((N, D), jnp.float32)
_grid = (N // TBLK,)

_t0 = pl.pallas_call(
    _t0_body, grid=_grid,
    in_specs=[_row_spec, _w_spec, _b_spec],
    out_specs=_row_spec, out_shape=_out_row)

_layer = pl.pallas_call(
    _layer_body, grid=_grid,
    in_specs=[_row_spec, _row_spec, _c_spec, _c_spec, _row_spec,
              _w_spec, _b_spec, _w_spec],
    out_specs=_row_spec, out_shape=_out_row)

_layer_final = pl.pallas_call(
    _layer_final_body, grid=_grid,
    in_specs=[_row_spec, _row_spec, _c_spec, _c_spec, _row_spec,
              _w_spec, _b_spec, _w_spec, _w_spec, _b_spec],
    out_specs=_row_spec, out_shape=_out_row)


def kernel(x, edge_index, W0, b0, Wl1, bl1, Wr1, Wl2, bl2, Wr2, W1, b1):
    src = edge_index[0]
    dst = edge_index[1]
    b0r = b0.reshape(1, D)
    bl1r = bl1.reshape(1, D)
    bl2r = bl2.reshape(1, D)
    b1r = b1.reshape(1, D)

    h0 = _t0(x, W0, b0r)
    part1, cnt = _seg_with_counts(h0, src, dst)
    cnt2 = cnt.reshape(NC, NPAD)
    c0 = cnt2[0, :N].reshape(N, 1)
    c1 = cnt2[1, :N].reshape(N, 1)
    h1 = _layer(part1[0, :N], part1[1, :N], c0, c1, h0, Wl1, bl1r, Wr1)
    (part2,) = _seg_no_counts(h1, src, dst)
    out = _layer_final(part2[0, :N], part2[1, :N], c0, c1, h1, Wl2, bl2r, Wr2,
                       W1, b1r)
    return out


# speedup vs baseline: 11.5645x; 1.0837x over previous
import functools

import jax
import jax.numpy as jnp
from jax import lax
from jax.experimental import pallas as pl
from jax.experimental.pallas import tpu as pltpu
import jax.experimental.pallas.tpu_sc as plsc

N = 10000
D = 128
E = 320000

NC = 2
NS = 16
NW = NC * NS
EPW = E // NW
CHUNK = 80
NCHUNK = EPW // CHUNK
NPAD = 10240
RPT = NPAD // NS
CPT = NPAD // NS


def _seg_body(with_counts, h_hbm, e_hbm, part_hbm, *rest):
    if with_counts:
        (cnt_hbm, src_v, dst_v, dstc_v, rbuf0, rbuf1, acc_s, cnt_s,
         ones_v, cbuf, gsem0, gsem1) = rest
    else:
        src_v, dst_v, dstc_v, rbuf0, rbuf1, acc_s, gsem0, gsem1 = rest
    c = lax.axis_index("c")
    s = lax.axis_index("s")
    w = c * NS + s

    pltpu.sync_copy(e_hbm.at[pl.ds(w * EPW, EPW)], src_v)
    pltpu.sync_copy(e_hbm.at[pl.ds(E + w * EPW, EPW)], dst_v)

    def _zrow(i, carry):
        def _zlane(j, carry2):
            rbuf0[i, pl.ds(j * 16, 16)] = jnp.zeros((16,), jnp.float32)
            return carry2
        return lax.fori_loop(0, D // 16, _zlane, carry)
    lax.fori_loop(0, CHUNK, _zrow, 0)
    for t in range(RPT // CHUNK):
        pltpu.sync_copy(rbuf0, acc_s.at[pl.ds(s * RPT + t * CHUNK, CHUNK), :])

    if with_counts:
        def _zc(i, carry):
            cbuf[pl.ds(i * 16, 16)] = jnp.zeros((16,), jnp.float32)
            return carry
        lax.fori_loop(0, CPT // 16, _zc, 0)
        pltpu.sync_copy(cbuf, cnt_s.at[pl.ds(s * CPT, CPT)])
        for j in range(CHUNK // 16):
            ones_v[pl.ds(j * 16, 16)] = jnp.ones((16,), jnp.float32)

    plsc.subcore_barrier()

    def _start(i, rb, sem):
        pltpu.async_copy(h_hbm.at[src_v.at[pl.ds(i * CHUNK, CHUNK)]], rb, sem)

    def _consume(i, rb, sem, prefetch_i):
        pltpu.make_async_copy(h_hbm.at[src_v.at[pl.ds(0, CHUNK)]], rb,
                              sem).wait()
        for k in range(CHUNK // 16):
            dstc_v[pl.ds(k * 16, 16)] = dst_v[pl.ds(i * CHUNK + k * 16, 16)]
        pltpu.sync_copy(rb, acc_s.at[dstc_v], add=True)
        if with_counts:
            pltpu.sync_copy(ones_v, cnt_s.at[dstc_v], add=True)

        @pl.when(prefetch_i < NCHUNK)
        def _():
            _start(prefetch_i, rb, sem)

    _start(0, rbuf0, gsem0)
    _start(1, rbuf1, gsem1)

    def _edge2(k, carry):
        i = 2 * k
        _consume(i, rbuf0, gsem0, i + 2)
        _consume(i + 1, rbuf1, gsem1, i + 3)
        return carry
    lax.fori_loop(0, NCHUNK // 2, _edge2, 0)
    _consume(NCHUNK - 1, rbuf0, gsem0, NCHUNK)

    plsc.subcore_barrier()

    for t in range(RPT // CHUNK):
        pltpu.sync_copy(acc_s.at[pl.ds(s * RPT + t * CHUNK, CHUNK), :], rbuf0)
        pltpu.sync_copy(rbuf0,
                        part_hbm.at[c, pl.ds(s * RPT + t * CHUNK, CHUNK), :])
    if with_counts:
        pltpu.sync_copy(cnt_s.at[pl.ds(s * CPT, CPT)], cbuf)
        pltpu.sync_copy(cbuf, cnt_hbm.at[pl.ds(c * NPAD + s * CPT, CPT)])


def _make_seg(with_counts):
    out_type = [jax.ShapeDtypeStruct((NC, NPAD, D), jnp.float32)]
    scratch = [
        pltpu.VMEM((EPW,), jnp.int32),
        pltpu.VMEM((EPW,), jnp.int32),
        pltpu.VMEM((CHUNK,), jnp.int32),
        pltpu.VMEM((CHUNK, D), jnp.float32),
        pltpu.VMEM((CHUNK, D), jnp.float32),
        pltpu.VMEM_SHARED((NPAD, D), jnp.float32),
    ]
    if with_counts:
        out_type.append(jax.ShapeDtypeStruct((NC * NPAD,), jnp.float32))
        scratch += [
            pltpu.VMEM_SHARED((NPAD,), jnp.float32),
            pltpu.VMEM((CHUNK,), jnp.float32),
            pltpu.VMEM((CPT,), jnp.float32),
        ]
    scratch.append(pltpu.SemaphoreType.DMA)
    scratch.append(pltpu.SemaphoreType.DMA)
    return pl.kernel(
        functools.partial(_seg_body, with_counts),
        out_type=out_type,
        mesh=plsc.VectorSubcoreMesh(core_axis_name="c", subcore_axis_name="s"),
        scratch_types=scratch,
    )


_seg_with_counts = _make_seg(True)
_seg_no_counts = _make_seg(False)


TBLK = 1024


def _t0_body(x_ref, w_ref, b_ref, o_ref):
    o_ref[...] = jnp.maximum(
        jnp.dot(x_ref[...], w_ref[...], preferred_element_type=jnp.float32)
        + b_ref[...], 0.0)


def _mean_agg(p0_ref, p1_ref, c0_ref, c1_ref):
    cnt = jnp.maximum(c0_ref[...] + c1_ref[...], 1.0)
    return (p0_ref[0] + p1_ref[0]) / cnt


def _layer_body(p0_ref, p1_ref, c0_ref, c1_ref, h_ref, wl_ref, bl_ref,
                wr_ref, o_ref):
    agg = _mean_agg(p0_ref, p1_ref, c0_ref, c1_ref)
    o_ref[...] = jnp.maximum(
        jnp.dot(agg, wl_ref[...], preferred_element_type=jnp.float32)
        + bl_ref[...]
        + jnp.dot(h_ref[...], wr_ref[...], preferred_element_type=jnp.float32),
        0.0)


def _layer_final_body(p0_ref, p1_ref, c0_ref, c1_ref, h_ref, wl_ref, bl_ref,
                      wr_ref, w1_ref, b1_ref, o_ref):
    agg = _mean_agg(p0_ref, p1_ref, c0_ref, c1_ref)
    h2 = jnp.maximum(
        jnp.dot(agg, wl_ref[...], preferred_element_type=jnp.float32)
        + bl_ref[...]
        + jnp.dot(h_ref[...], wr_ref[...], preferred_element_type=jnp.float32),
        0.0)
    o_ref[...] = (jnp.dot(h2, w1_ref[...], preferred_element_type=jnp.float32)
                  + b1_ref[...])


_row_spec = pl.BlockSpec((TBLK, D), lambda i: (i, 0))
_p0_spec = pl.BlockSpec((1, TBLK, D), lambda i: (0, i, 0))
_p1_spec = pl.BlockSpec((1, TBLK, D), lambda i: (1, i, 0))
_w_spec = pl.BlockSpec((D, D), lambda i: (0, 0))
_b_spec = pl.BlockSpec((1, D), lambda i: (0, 0))
_c_spec = pl.BlockSpec((TBLK, 1), lambda i: (i, 0))
_out_pad = jax.ShapeDtypeStruct((NPAD, D), jnp.float32)
_out_n = jax.ShapeDtypeStruct((N, D), jnp.float32)
_grid = (NPAD // TBLK,)

_t0 = pl.pallas_call(
    _t0_body, grid=_grid,
    in_specs=[_row_spec, _w_spec, _b_spec],
    out_specs=_row_spec, out_shape=_out_pad)

_layer = pl.pallas_call(
    _layer_body, grid=_grid,
    in_specs=[_p0_spec, _p1_spec, _c_spec, _c_spec, _row_spec,
              _w_spec, _b_spec, _w_spec],
    out_specs=_row_spec, out_shape=_out_pad)

_layer_final = pl.pallas_call(
    _layer_final_body, grid=_grid,
    in_specs=[_p0_spec, _p1_spec, _c_spec, _c_spec, _row_spec,
              _w_spec, _b_spec, _w_spec, _w_spec, _b_spec],
    out_specs=_row_spec, out_shape=_out_n)


def kernel(x, edge_index, W0, b0, Wl1, bl1, Wr1, Wl2, bl2, Wr2, W1, b1):
    eflat = edge_index.reshape(2 * E)
    b0r = b0.reshape(1, D)
    bl1r = bl1.reshape(1, D)
    bl2r = bl2.reshape(1, D)
    b1r = b1.reshape(1, D)

    h0 = _t0(x, W0, b0r)
    part1, cnt = _seg_with_counts(h0, eflat)
    c0 = cnt[:NPAD].reshape(NPAD, 1)
    c1 = cnt[NPAD:].reshape(NPAD, 1)
    h1 = _layer(part1, part1, c0, c1, h0, Wl1, bl1r, Wr1)
    (part2,) = _seg_no_counts(h1, eflat)
    out = _layer_final(part2, part2, c0, c1, h1, Wl2, bl2r, Wr2, W1, b1r)
    return out
